# Initial kernel scaffold; baseline (speedup 1.0000x reference)
import jax
import jax.numpy as jnp
from jax.experimental import pallas as pl


def kernel(h, ef, dt, time_w, time_b, Wq, bq, sq, Wkv, bkv, skv, Wwq, bwq, swq, Wwk, bwk, swk, Wwv, bwv, swv, att_scale, edge_dst):
    raise NotImplementedError("write your pallas kernel here")


# speedup vs baseline: 2.0925x; 2.0925x over previous
import functools

import jax
import jax.numpy as jnp
from jax import lax
from jax.experimental import pallas as pl
from jax.experimental.pallas import tpu as pltpu
from jax.experimental.pallas import tpu_sc as plsc

NC = 2
NS = 16
NW = NC * NS
CH = 128

PW = 128


def _leaky(x):
  return jnp.where(x >= 0, x, 0.2 * x)


def _lorentz_scale(pre, es, col, neg_time=False):
  t0 = pre[:, 0:1]
  time = jax.nn.sigmoid(t0) * es + 1.1
  s2 = jnp.sum(pre * pre, axis=1, keepdims=True) - t0 * t0
  scale = (time * time - 1.0) / jnp.maximum(s2, 1e-8)
  root = jnp.sqrt(scale)
  tcol = -time if neg_time else time
  return jnp.where(col == 0, tcol, pre * root)


def _node_kernel(h_ref, tb_ref, wqt_h_ref, wqt_tf_ref, wq_c0_ref, bq_ref,
                 wwqt_ref, bwq_ref, wwkt_ref, bwk_ref, wwvt_ref, bwv_ref,
                 sc_ref, qt_ref, pself_ref):
  es_q = sc_ref[0, 0]
  es_wq = sc_ref[0, 1]
  es_wk = sc_ref[0, 2]
  es_wv = sc_ref[0, 3]
  inv_s = sc_ref[0, 4]

  hb = h_ref[...]
  n = hb.shape[0]
  col = lax.broadcasted_iota(jnp.int32, (n, 128), 1)

  ztf = jnp.cos(tb_ref[...])
  zterm = jnp.dot(_leaky(ztf), wqt_tf_ref[...],
                  precision=lax.Precision.HIGHEST, preferred_element_type=jnp.float32)
  z2 = jnp.sum(ztf * ztf, axis=1, keepdims=True)

  sq = jnp.sum(hb * hb, axis=1, keepdims=True) + z2
  xn = jnp.maximum(jnp.sqrt(sq), 1e-8)
  en = jnp.exp(xn)
  eni = 1.0 / en
  sinh = 0.5 * (en - eni)
  cosh = 0.5 * (en + eni)
  coef = sinh / xn

  pre = (coef * (jnp.dot(_leaky(hb), wqt_h_ref[...],
                         precision=lax.Precision.HIGHEST, preferred_element_type=jnp.float32) + zterm)
         + cosh * wq_c0_ref[...] + bq_ref[...])
  q_ori = _lorentz_scale(pre, es_q, col)

  qw_pre = jnp.dot(q_ori, wwqt_ref[...],
                   precision=lax.Precision.HIGHEST, preferred_element_type=jnp.float32) + bwq_ref[...]
  qt = _lorentz_scale(qw_pre, es_wq, col, neg_time=True)

  kc_pre = jnp.dot(q_ori, wwkt_ref[...],
                   precision=lax.Precision.HIGHEST, preferred_element_type=jnp.float32) + bwk_ref[...]
  kcs = _lorentz_scale(kc_pre, es_wk, col)

  v_pre = jnp.dot(kcs, wwvt_ref[...],
                  precision=lax.Precision.HIGHEST, preferred_element_type=jnp.float32) + bwv_ref[...]
  vs = _lorentz_scale(v_pre, es_wv, col)

  ip = jnp.sum(qt * kcs, axis=1, keepdims=True)
  e = jnp.exp((2.0 + 2.0 * ip) * inv_s)

  qt_ref[...] = qt
  pself_ref[...] = e * vs


def _edge_kernel(h_ref, ef_ref, dt_ref, qg_ref, tw_ref, tb_ref,
                 wkvt_h_ref, wkvt_ef_ref, wkvt_tf_ref, wkv_c0_ref, bkv_ref,
                 wwkt_ref, bwk_ref, wwvt_ref, bwv_ref, sc_ref, p_ref):
  es_kv = sc_ref[0, 0]
  es_wk = sc_ref[0, 1]
  es_wv = sc_ref[0, 2]
  inv_s = sc_ref[0, 3]

  hb = h_ref[...]
  efb = ef_ref[...]
  n = hb.shape[0]
  col = lax.broadcasted_iota(jnp.int32, (n, 128), 1)

  tf = jnp.cos(dt_ref[...] * tw_ref[...] + tb_ref[...])

  sq = (jnp.sum(hb * hb, axis=1, keepdims=True)
        + jnp.sum(efb * efb, axis=1, keepdims=True)
        + jnp.sum(tf * tf, axis=1, keepdims=True))
  xn = jnp.maximum(jnp.sqrt(sq), 1e-8)
  en = jnp.exp(xn)
  eni = 1.0 / en
  sinh = 0.5 * (en - eni)
  cosh = 0.5 * (en + eni)
  coef = sinh / xn

  mm = (jnp.dot(_leaky(hb), wkvt_h_ref[...],
                precision=lax.Precision.HIGHEST, preferred_element_type=jnp.float32)
        + jnp.dot(_leaky(efb), wkvt_ef_ref[...],
                  precision=lax.Precision.HIGHEST, preferred_element_type=jnp.float32)
        + jnp.dot(_leaky(tf), wkvt_tf_ref[...],
                  precision=lax.Precision.HIGHEST, preferred_element_type=jnp.float32))
  pre = coef * mm + cosh * wkv_c0_ref[...] + bkv_ref[...]
  k_ori = _lorentz_scale(pre, es_kv, col)

  kc_pre = jnp.dot(k_ori, wwkt_ref[...],
                   precision=lax.Precision.HIGHEST, preferred_element_type=jnp.float32) + bwk_ref[...]
  kc = _lorentz_scale(kc_pre, es_wk, col)

  v_pre = jnp.dot(kc, wwvt_ref[...],
                  precision=lax.Precision.HIGHEST, preferred_element_type=jnp.float32) + bwv_ref[...]
  v = _lorentz_scale(v_pre, es_wv, col)

  ip = jnp.sum(qg_ref[...] * kc, axis=1, keepdims=True)
  e = jnp.exp((2.0 + 2.0 * ip) * inv_s)

  p_ref[...] = e * v


def _finish_kernel(acc_a_ref, acc_b_ref, pself_ref, out_ref):
  r = acc_a_ref[...] + acc_b_ref[...] + pself_ref[...]
  n = r.shape[0]
  col = lax.broadcasted_iota(jnp.int32, (n, 128), 1)

  t0 = r[:, 0:1]
  ip = jnp.sum(r * r, axis=1, keepdims=True) - 2.0 * t0 * t0
  denom = jnp.sqrt(jnp.maximum(jnp.abs(ip), 1e-30))
  rst = r / denom

  r0 = rst[:, 0:1]
  tcl = jnp.maximum(r0, 1.0 + 1e-7)
  d = jnp.log(tcl + jnp.sqrt(tcl * tcl - 1.0))
  s2 = jnp.maximum(jnp.sum(rst * rst, axis=1, keepdims=True) - r0 * r0, 0.0)
  yn = jnp.maximum(jnp.sqrt(s2), 1e-8)
  out_ref[...] = jnp.where(col == 0, 0.0, rst * (d / yn))


def _sc_gather_body(qtab, idx2d, qg, idx_v, rows_v, sem):
  cid = lax.axis_index("c")
  sid = lax.axis_index("s")
  g = sid * NC + cid
  nch = idx2d.shape[0]
  base = nch // NW
  rem = nch - base * NW

  def do_chunk(c):
    pltpu.sync_copy(idx2d.at[c], idx_v)
    pltpu.async_copy(qtab.at[idx_v], rows_v, sem).wait()
    pltpu.sync_copy(rows_v, qg.at[pl.ds(pl.multiple_of(c * CH, CH), CH)])

  def body(j, carry):
    do_chunk(j * NW + g)
    return carry

  lax.fori_loop(0, base, body, 0)

  @pl.when(g < rem)
  def _():
    do_chunk(base * NW + g)


def _sc_scatter_body(p, idx2d, zrows, acc_a, acc_b, idx_v, rows_v, shared):
  cid = lax.axis_index("c")
  sid = lax.axis_index("s")
  g = sid * NC + cid
  nch = idx2d.shape[0]
  base = nch // NW
  rem = nch - base * NW
  rows_per_tile = shared.shape[0] // NS

  r0 = pl.multiple_of(sid * rows_per_tile, CH)
  pltpu.sync_copy(zrows, rows_v)
  for off in range(0, rows_per_tile, CH):
    pltpu.sync_copy(rows_v, shared.at[pl.ds(r0 + off, CH)])
  plsc.subcore_barrier()

  def do_chunk(c):
    pltpu.sync_copy(idx2d.at[c], idx_v)
    pltpu.sync_copy(p.at[pl.ds(pl.multiple_of(c * CH, CH), CH)], rows_v)
    pltpu.sync_copy(rows_v, shared.at[idx_v], add=True)

  def body(j, carry):
    do_chunk(j * NW + g)
    return carry

  lax.fori_loop(0, base, body, 0)

  @pl.when(g < rem)
  def _():
    do_chunk(base * NW + g)

  plsc.subcore_barrier()

  for off in range(0, rows_per_tile, CH):
    pltpu.sync_copy(shared.at[pl.ds(r0 + off, CH)], rows_v)

    @pl.when(cid == 0)
    def _():
      pltpu.sync_copy(rows_v, acc_a.at[pl.ds(r0 + off, CH)])

    @pl.when(cid == 1)
    def _():
      pltpu.sync_copy(rows_v, acc_b.at[pl.ds(r0 + off, CH)])


def kernel(h, ef, dt, time_w, time_b, Wq, bq, sq, Wkv, bkv, skv,
           Wwq, bwq, swq, Wwk, bwk, swk, Wwv, bwv, swv, att_scale, edge_dst):
  E = edge_dst.shape[0]
  D = h.shape[0] - E
  f32 = jnp.float32

  BN = 400
  BE = 400
  n_node_blocks = D // BN
  n_edge_blocks = E // BE

  tw = time_w.reshape(1, -1)
  tb = time_b.reshape(1, -1)
  dt2 = dt.reshape(E, 1)
  idx2d = edge_dst.astype(jnp.int32).reshape(E // CH, CH)

  WqT = Wq.T
  wqt_h = WqT[1:129]
  wqt_tf = WqT[129:229]
  wq_c0 = WqT[0:1]
  WkvT = Wkv.T
  wkvt_h = WkvT[1:129]
  wkvt_ef = WkvT[129:145]
  wkvt_tf = WkvT[145:245]
  wkv_c0 = WkvT[0:1]
  wwqt = Wwq.T
  wwkt = Wwk.T
  wwvt = Wwv.T
  bq2 = bq.reshape(1, -1)
  bkv2 = bkv.reshape(1, -1)
  bwq2 = bwq.reshape(1, -1)
  bwk2 = bwk.reshape(1, -1)
  bwv2 = bwv.reshape(1, -1)
  inv_s = 1.0 / att_scale
  sc_node = jnp.stack([jnp.exp(sq), jnp.exp(swq), jnp.exp(swk),
                       jnp.exp(swv), inv_s]).reshape(1, 5).astype(f32)
  sc_edge = jnp.stack([jnp.exp(skv), jnp.exp(swk), jnp.exp(swv),
                       inv_s]).reshape(1, 4).astype(f32)
  zrows = jnp.zeros((CH, PW), f32)

  full = lambda *shape: pl.BlockSpec(shape, lambda i: tuple(0 for _ in shape))

  qt, pself = pl.pallas_call(
      _node_kernel,
      grid=(n_node_blocks,),
      in_specs=[
          pl.BlockSpec((BN, 128), lambda i: (i, 0)),
          full(1, 100),
          full(128, 128), full(100, 128), full(1, 128),
          full(1, 128),
          full(128, 128), full(1, 128),
          full(128, 128), full(1, 128),
          full(128, 128), full(1, 128),
          full(1, 5),
      ],
      out_specs=[
          pl.BlockSpec((BN, 128), lambda i: (i, 0)),
          pl.BlockSpec((BN, 128), lambda i: (i, 0)),
      ],
      out_shape=[
          jax.ShapeDtypeStruct((D, 128), f32),
          jax.ShapeDtypeStruct((D, 128), f32),
      ],
  )(h, tb, wqt_h, wqt_tf, wq_c0, bq2, wwqt, bwq2, wwkt, bwk2, wwvt, bwv2,
    sc_node)

  mesh = plsc.VectorSubcoreMesh(core_axis_name="c", subcore_axis_name="s",
                                num_cores=NC, num_subcores=NS)
  qg = pl.kernel(
      _sc_gather_body,
      out_type=jax.ShapeDtypeStruct((E, 128), f32),
      mesh=mesh,
      scratch_types=[
          pltpu.VMEM((CH,), jnp.int32),
          pltpu.VMEM((CH, 128), f32),
          pltpu.SemaphoreType.DMA,
      ],
  )(qt, idx2d)

  p_rows = pl.pallas_call(
      _edge_kernel,
      grid=(n_edge_blocks,),
      in_specs=[
          pl.BlockSpec((BE, 128), lambda i: (i + n_node_blocks, 0)),
          pl.BlockSpec((BE, 16), lambda i: (i, 0)),
          pl.BlockSpec((BE, 1), lambda i: (i, 0)),
          pl.BlockSpec((BE, 128), lambda i: (i, 0)),
          full(1, 100), full(1, 100),
          full(128, 128), full(16, 128), full(100, 128),
          full(1, 128), full(1, 128),
          full(128, 128), full(1, 128),
          full(128, 128), full(1, 128),
          full(1, 4),
      ],
      out_specs=pl.BlockSpec((BE, PW), lambda i: (i, 0)),
      out_shape=jax.ShapeDtypeStruct((E, PW), f32),
  )(h, ef, dt2, qg, tw, tb, wkvt_h, wkvt_ef, wkvt_tf, wkv_c0, bkv2,
    wwkt, bwk2, wwvt, bwv2, sc_edge)

  d_pad = ((D + NS * CH - 1) // (NS * CH)) * (NS * CH)
  acc_a, acc_b = pl.kernel(
      _sc_scatter_body,
      out_type=[jax.ShapeDtypeStruct((d_pad, PW), f32),
                jax.ShapeDtypeStruct((d_pad, PW), f32)],
      mesh=mesh,
      scratch_types=[
          pltpu.VMEM((CH,), jnp.int32),
          pltpu.VMEM((CH, PW), f32),
          pltpu.VMEM_SHARED((d_pad, PW), f32),
      ],
  )(p_rows, idx2d, zrows)

  out = pl.pallas_call(
      _finish_kernel,
      grid=(n_node_blocks,),
      in_specs=[
          pl.BlockSpec((BN, PW), lambda i: (i, 0)),
          pl.BlockSpec((BN, PW), lambda i: (i, 0)),
          pl.BlockSpec((BN, 128), lambda i: (i, 0)),
      ],
      out_specs=pl.BlockSpec((BN, 128), lambda i: (i, 0)),
      out_shape=jax.ShapeDtypeStruct((D, 128), f32),
  )(acc_a, acc_b, pself)

  return out


# speedup vs baseline: 2.0978x; 1.0025x over previous
import functools

import jax
import jax.numpy as jnp
from jax import lax
from jax.experimental import pallas as pl
from jax.experimental.pallas import tpu as pltpu
from jax.experimental.pallas import tpu_sc as plsc

NC = 2
NS = 16
NW = NC * NS
CH = 128

PW = 128


def _leaky(x):
  return jnp.maximum(x, 0.2 * x)


def _lorentz_scale(pre, es, col, neg_time=False):
  t0 = pre[:, 0:1]
  time = jax.nn.sigmoid(t0) * es + 1.1
  s2 = jnp.sum(pre * pre, axis=1, keepdims=True) - t0 * t0
  scale = (time * time - 1.0) / jnp.maximum(s2, 1e-8)
  root = jnp.sqrt(scale)
  tcol = -time if neg_time else time
  return jnp.where(col == 0, tcol, pre * root)


def _node_kernel(h_ref, tb_ref, wqt_h_ref, wqt_tf_ref, wq_c0_ref, bq_ref,
                 wwqt_ref, bwq_ref, wwkt_ref, bwk_ref, wwvt_ref, bwv_ref,
                 sc_ref, qt_ref, pself_ref):
  es_q = sc_ref[0, 0]
  es_wq = sc_ref[0, 1]
  es_wk = sc_ref[0, 2]
  es_wv = sc_ref[0, 3]
  inv_s = sc_ref[0, 4]

  hb = h_ref[...]
  n = hb.shape[0]
  col = lax.broadcasted_iota(jnp.int32, (n, 128), 1)

  ztf = jnp.cos(tb_ref[...])
  zterm = jnp.dot(_leaky(ztf), wqt_tf_ref[...],
                  precision=lax.Precision.HIGHEST, preferred_element_type=jnp.float32)
  z2 = jnp.sum(ztf * ztf, axis=1, keepdims=True)

  sq = jnp.sum(hb * hb, axis=1, keepdims=True) + z2
  xn = jnp.maximum(jnp.sqrt(sq), 1e-8)
  en = jnp.exp(xn)
  eni = 1.0 / en
  sinh = 0.5 * (en - eni)
  cosh = 0.5 * (en + eni)
  coef = sinh / xn

  pre = (coef * (jnp.dot(_leaky(hb), wqt_h_ref[...],
                         precision=lax.Precision.HIGHEST, preferred_element_type=jnp.float32) + zterm)
         + cosh * wq_c0_ref[...] + bq_ref[...])
  q_ori = _lorentz_scale(pre, es_q, col)

  qw_pre = jnp.dot(q_ori, wwqt_ref[...],
                   precision=lax.Precision.HIGHEST, preferred_element_type=jnp.float32) + bwq_ref[...]
  qt = _lorentz_scale(qw_pre, es_wq, col, neg_time=True)

  kc_pre = jnp.dot(q_ori, wwkt_ref[...],
                   precision=lax.Precision.HIGHEST, preferred_element_type=jnp.float32) + bwk_ref[...]
  kcs = _lorentz_scale(kc_pre, es_wk, col)

  v_pre = jnp.dot(kcs, wwvt_ref[...],
                  precision=lax.Precision.HIGHEST, preferred_element_type=jnp.float32) + bwv_ref[...]
  vs = _lorentz_scale(v_pre, es_wv, col)

  ip = jnp.sum(qt * kcs, axis=1, keepdims=True)
  e = jnp.exp((2.0 + 2.0 * ip) * inv_s)

  qt_ref[...] = qt
  pself_ref[...] = e * vs


def _edge_kernel(h_ref, ef_ref, dt_ref, qg_ref, tw_ref, tb_ref,
                 wkvt_h_ref, wkvt_ef_ref, wkvt_tf_ref, wkv_c0_ref, bkv_ref,
                 wwkt_ref, bwk_ref, wwvt_ref, bwv_ref, sc_ref, p_ref):
  es_kv = sc_ref[0, 0]
  es_wk = sc_ref[0, 1]
  es_wv = sc_ref[0, 2]
  inv_s = sc_ref[0, 3]

  hb = h_ref[...]
  efb = ef_ref[...]
  n = hb.shape[0]
  col = lax.broadcasted_iota(jnp.int32, (n, 128), 1)

  tf = jnp.cos(dt_ref[...] * tw_ref[...] + tb_ref[...])

  sq = (jnp.sum(hb * hb, axis=1, keepdims=True)
        + jnp.sum(efb * efb, axis=1, keepdims=True)
        + jnp.sum(tf * tf, axis=1, keepdims=True))
  xn = jnp.maximum(jnp.sqrt(sq), 1e-8)
  en = jnp.exp(xn)
  eni = 1.0 / en
  sinh = 0.5 * (en - eni)
  cosh = 0.5 * (en + eni)
  coef = sinh / xn

  mm = (jnp.dot(_leaky(hb), wkvt_h_ref[...],
                precision=lax.Precision.HIGHEST, preferred_element_type=jnp.float32)
        + jnp.dot(_leaky(efb), wkvt_ef_ref[...],
                  precision=lax.Precision.HIGHEST, preferred_element_type=jnp.float32)
        + jnp.dot(_leaky(tf), wkvt_tf_ref[...],
                  precision=lax.Precision.HIGHEST, preferred_element_type=jnp.float32))
  pre = coef * mm + cosh * wkv_c0_ref[...] + bkv_ref[...]
  k_ori = _lorentz_scale(pre, es_kv, col)

  kc_pre = jnp.dot(k_ori, wwkt_ref[...],
                   precision=lax.Precision.HIGHEST, preferred_element_type=jnp.float32) + bwk_ref[...]
  kc = _lorentz_scale(kc_pre, es_wk, col)

  v_pre = jnp.dot(kc, wwvt_ref[...],
                  precision=lax.Precision.HIGHEST, preferred_element_type=jnp.float32) + bwv_ref[...]
  v = _lorentz_scale(v_pre, es_wv, col)

  ip = jnp.sum(qg_ref[...] * kc, axis=1, keepdims=True)
  e = jnp.exp((2.0 + 2.0 * ip) * inv_s)

  p_ref[...] = e * v


def _finish_kernel(acc_a_ref, acc_b_ref, pself_ref, out_ref):
  r = acc_a_ref[...] + acc_b_ref[...] + pself_ref[...]
  n = r.shape[0]
  col = lax.broadcasted_iota(jnp.int32, (n, 128), 1)

  t0 = r[:, 0:1]
  ip = jnp.sum(r * r, axis=1, keepdims=True) - 2.0 * t0 * t0
  denom = jnp.sqrt(jnp.maximum(jnp.abs(ip), 1e-30))
  rst = r / denom

  r0 = rst[:, 0:1]
  tcl = jnp.maximum(r0, 1.0 + 1e-7)
  d = jnp.log(tcl + jnp.sqrt(tcl * tcl - 1.0))
  s2 = jnp.maximum(jnp.sum(rst * rst, axis=1, keepdims=True) - r0 * r0, 0.0)
  yn = jnp.maximum(jnp.sqrt(s2), 1e-8)
  out_ref[...] = jnp.where(col == 0, 0.0, rst * (d / yn))


def _sc_gather_body(qtab, idx2d, qg, idx_v, rows_v, sem):
  cid = lax.axis_index("c")
  sid = lax.axis_index("s")
  g = sid * NC + cid
  nch = idx2d.shape[0]
  base = nch // NW
  rem = nch - base * NW

  def do_chunk(c):
    pltpu.sync_copy(idx2d.at[c], idx_v)
    pltpu.async_copy(qtab.at[idx_v], rows_v, sem).wait()
    pltpu.sync_copy(rows_v, qg.at[pl.ds(pl.multiple_of(c * CH, CH), CH)])

  def body(j, carry):
    do_chunk(j * NW + g)
    return carry

  lax.fori_loop(0, base, body, 0)

  @pl.when(g < rem)
  def _():
    do_chunk(base * NW + g)


def _sc_scatter_body(p, idx2d, zrows, acc_a, acc_b, idx_v, rows_v, shared):
  cid = lax.axis_index("c")
  sid = lax.axis_index("s")
  g = sid * NC + cid
  nch = idx2d.shape[0]
  base = nch // NW
  rem = nch - base * NW
  rows_per_tile = shared.shape[0] // NS

  r0 = pl.multiple_of(sid * rows_per_tile, CH)
  pltpu.sync_copy(zrows, rows_v)
  for off in range(0, rows_per_tile, CH):
    pltpu.sync_copy(rows_v, shared.at[pl.ds(r0 + off, CH)])
  plsc.subcore_barrier()

  def do_chunk(c):
    pltpu.sync_copy(idx2d.at[c], idx_v)
    pltpu.sync_copy(p.at[pl.ds(pl.multiple_of(c * CH, CH), CH)], rows_v)
    pltpu.sync_copy(rows_v, shared.at[idx_v], add=True)

  def body(j, carry):
    do_chunk(j * NW + g)
    return carry

  lax.fori_loop(0, base, body, 0)

  @pl.when(g < rem)
  def _():
    do_chunk(base * NW + g)

  plsc.subcore_barrier()

  for off in range(0, rows_per_tile, CH):
    pltpu.sync_copy(shared.at[pl.ds(r0 + off, CH)], rows_v)

    @pl.when(cid == 0)
    def _():
      pltpu.sync_copy(rows_v, acc_a.at[pl.ds(r0 + off, CH)])

    @pl.when(cid == 1)
    def _():
      pltpu.sync_copy(rows_v, acc_b.at[pl.ds(r0 + off, CH)])


def kernel(h, ef, dt, time_w, time_b, Wq, bq, sq, Wkv, bkv, skv,
           Wwq, bwq, swq, Wwk, bwk, swk, Wwv, bwv, swv, att_scale, edge_dst):
  E = edge_dst.shape[0]
  D = h.shape[0] - E
  f32 = jnp.float32

  BN = 2000
  BE = 2000
  n_node_blocks = D // BN
  n_edge_blocks = E // BE

  tw = time_w.reshape(1, -1)
  tb = time_b.reshape(1, -1)
  dt2 = dt.reshape(E, 1)
  idx2d = edge_dst.astype(jnp.int32).reshape(E // CH, CH)

  WqT = Wq.T
  wqt_h = WqT[1:129]
  wqt_tf = WqT[129:229]
  wq_c0 = WqT[0:1]
  WkvT = Wkv.T
  wkvt_h = WkvT[1:129]
  wkvt_ef = WkvT[129:145]
  wkvt_tf = WkvT[145:245]
  wkv_c0 = WkvT[0:1]
  wwqt = Wwq.T
  wwkt = Wwk.T
  wwvt = Wwv.T
  bq2 = bq.reshape(1, -1)
  bkv2 = bkv.reshape(1, -1)
  bwq2 = bwq.reshape(1, -1)
  bwk2 = bwk.reshape(1, -1)
  bwv2 = bwv.reshape(1, -1)
  inv_s = 1.0 / att_scale
  sc_node = jnp.stack([jnp.exp(sq), jnp.exp(swq), jnp.exp(swk),
                       jnp.exp(swv), inv_s]).reshape(1, 5).astype(f32)
  sc_edge = jnp.stack([jnp.exp(skv), jnp.exp(swk), jnp.exp(swv),
                       inv_s]).reshape(1, 4).astype(f32)
  zrows = jnp.zeros((CH, PW), f32)

  full = lambda *shape: pl.BlockSpec(shape, lambda i: tuple(0 for _ in shape))

  qt, pself = pl.pallas_call(
      _node_kernel,
      grid=(n_node_blocks,),
      in_specs=[
          pl.BlockSpec((BN, 128), lambda i: (i, 0)),
          full(1, 100),
          full(128, 128), full(100, 128), full(1, 128),
          full(1, 128),
          full(128, 128), full(1, 128),
          full(128, 128), full(1, 128),
          full(128, 128), full(1, 128),
          full(1, 5),
      ],
      out_specs=[
          pl.BlockSpec((BN, 128), lambda i: (i, 0)),
          pl.BlockSpec((BN, 128), lambda i: (i, 0)),
      ],
      out_shape=[
          jax.ShapeDtypeStruct((D, 128), f32),
          jax.ShapeDtypeStruct((D, 128), f32),
      ],
  )(h, tb, wqt_h, wqt_tf, wq_c0, bq2, wwqt, bwq2, wwkt, bwk2, wwvt, bwv2,
    sc_node)

  mesh = plsc.VectorSubcoreMesh(core_axis_name="c", subcore_axis_name="s",
                                num_cores=NC, num_subcores=NS)
  qg = pl.kernel(
      _sc_gather_body,
      out_type=jax.ShapeDtypeStruct((E, 128), f32),
      mesh=mesh,
      scratch_types=[
          pltpu.VMEM((CH,), jnp.int32),
          pltpu.VMEM((CH, 128), f32),
          pltpu.SemaphoreType.DMA,
      ],
  )(qt, idx2d)

  p_rows = pl.pallas_call(
      _edge_kernel,
      grid=(n_edge_blocks,),
      in_specs=[
          pl.BlockSpec((BE, 128), lambda i: (i + n_node_blocks, 0)),
          pl.BlockSpec((BE, 16), lambda i: (i, 0)),
          pl.BlockSpec((BE, 1), lambda i: (i, 0)),
          pl.BlockSpec((BE, 128), lambda i: (i, 0)),
          full(1, 100), full(1, 100),
          full(128, 128), full(16, 128), full(100, 128),
          full(1, 128), full(1, 128),
          full(128, 128), full(1, 128),
          full(128, 128), full(1, 128),
          full(1, 4),
      ],
      out_specs=pl.BlockSpec((BE, PW), lambda i: (i, 0)),
      out_shape=jax.ShapeDtypeStruct((E, PW), f32),
  )(h, ef, dt2, qg, tw, tb, wkvt_h, wkvt_ef, wkvt_tf, wkv_c0, bkv2,
    wwkt, bwk2, wwvt, bwv2, sc_edge)

  d_pad = ((D + NS * CH - 1) // (NS * CH)) * (NS * CH)
  acc_a, acc_b = pl.kernel(
      _sc_scatter_body,
      out_type=[jax.ShapeDtypeStruct((d_pad, PW), f32),
                jax.ShapeDtypeStruct((d_pad, PW), f32)],
      mesh=mesh,
      scratch_types=[
          pltpu.VMEM((CH,), jnp.int32),
          pltpu.VMEM((CH, PW), f32),
          pltpu.VMEM_SHARED((d_pad, PW), f32),
      ],
  )(p_rows, idx2d, zrows)

  out = pl.pallas_call(
      _finish_kernel,
      grid=(n_node_blocks,),
      in_specs=[
          pl.BlockSpec((BN, PW), lambda i: (i, 0)),
          pl.BlockSpec((BN, PW), lambda i: (i, 0)),
          pl.BlockSpec((BN, 128), lambda i: (i, 0)),
      ],
      out_specs=pl.BlockSpec((BN, 128), lambda i: (i, 0)),
      out_shape=jax.ShapeDtypeStruct((D, 128), f32),
  )(acc_a, acc_b, pself)

  return out
